# baseline (speedup 1.0000x reference)
import functools

import jax
import jax.numpy as jnp
from jax import lax
from jax.experimental import pallas as pl
from jax.experimental.pallas import tpu as pltpu
from jax.experimental.pallas import tpu_sc as plsc

NC = 2
NS = 16
NW = NC * NS


def _type_gather(x, i, j, off, es):
    N = x.shape[0]
    EW = es // NW
    CA = 1000
    assert es % NW == 0 and EW % CA == 0 and CA % 8 == 0
    mesh = plsc.VectorSubcoreMesh(core_axis_name="c", subcore_axis_name="s")

    @functools.partial(
        pl.kernel,
        mesh=mesh,
        out_type=(
            jax.ShapeDtypeStruct((es,), jnp.int32),
            jax.ShapeDtypeStruct((es,), jnp.int32),
        ),
        scratch_types=[
            pltpu.VMEM_SHARED((N,), jnp.int32),
            pltpu.VMEM((CA,), jnp.int32),
            pltpu.VMEM((CA,), jnp.int32),
            pltpu.VMEM((CA,), jnp.int32),
            pltpu.VMEM((CA,), jnp.int32),
            pltpu.SemaphoreType.DMA,
            pltpu.SemaphoreType.DMA,
            pltpu.SemaphoreType.DMA,
            pltpu.SemaphoreType.DMA,
            pltpu.SemaphoreType.DMA,
            pltpu.SemaphoreType.DMA,
        ],
    )
    def k(x_hbm, i_hbm, j_hbm, xi_hbm, xj_hbm,
          xsh, ib0, ib1, ob0, ob1, si0, si1, sg0, sg1, so0, so1):
        s_idx = lax.axis_index("s")
        w = s_idx * NC + lax.axis_index("c")

        @pl.when(s_idx == 0)
        def _stage():
            pltpu.sync_copy(x_hbm, xsh)

        plsc.subcore_barrier()
        nch = EW // CA
        units = [(i_hbm, xi_hbm, kk) for kk in range(nch)]
        units += [(j_hbm, xj_hbm, kk) for kk in range(nch)]
        slots = ((ib0, ob0, si0, sg0, so0), (ib1, ob1, si1, sg1, so1))
        nu = len(units)
        pend_in, pend_out = {}, {}

        def issue_in(m):
            src, _, kk = units[m]
            ib, _, si, _, _ = slots[m % 2]
            base = off + w * EW + kk * CA
            pend_in[m] = pltpu.async_copy(src.at[pl.ds(base, CA)], ib, si)

        issue_in(0)
        issue_in(1)
        for m in range(nu):
            ib, ob, si, sg, so = slots[m % 2]
            _, dst, kk = units[m]
            base = w * EW + kk * CA
            pend_in[m].wait()
            if m >= 2:
                pend_out[m - 2].wait()
            pltpu.async_copy(xsh.at[ib], ob, sg).wait()
            pend_out[m] = pltpu.async_copy(ob, dst.at[pl.ds(base, CA)], so)
            if m + 2 < nu:
                issue_in(m + 2)
        pend_out[nu - 2].wait()
        pend_out[nu - 1].wait()

    return k(x, i, j)


def _edge_tc(xi, xj, rbf, embp, W, bias2, b2, blk_off, Eb):
    es = xi.shape[0]
    H = rbf.shape[1]
    assert es % Eb == 0
    nblk = es // Eb
    xi = xi.reshape(nblk, 1, Eb)
    xj = xj.reshape(nblk, 1, Eb)

    def body(xi_ref, xj_ref, rbf_ref, emb_ref, w_ref, bias_ref, b_ref,
             h_ref, t12, w3):
        @pl.when(pl.program_id(0) == 0)
        def _init():
            embv = emb_ref[...]
            dn = (((1,), (1,)), ((), ()))
            t12[0:H, :] = lax.dot_general(
                embv, w_ref[:, 0:H], dn, preferred_element_type=jnp.float32)
            t12[H:2 * H, :] = lax.dot_general(
                embv, w_ref[:, H:2 * H], dn, preferred_element_type=jnp.float32)
            w3[...] = w_ref[:, 2 * H:3 * H]

        xi_v = xi_ref[...].reshape(1, Eb)
        xj_v = xj_ref[...].reshape(1, Eb) + H
        iot = lax.broadcasted_iota(jnp.int32, (2 * H, Eb), 0)
        oht = jnp.where((iot == xi_v) | (iot == xj_v),
                        1.0, 0.0).astype(jnp.bfloat16)
        ra = rbf_ref[...] + bias_ref[...]
        ra = ra * jax.nn.sigmoid(ra)
        t = lax.dot_general(ra.astype(jnp.bfloat16),
                            w3[...].astype(jnp.bfloat16),
                            (((1,), (1,)), ((), ())),
                            preferred_element_type=jnp.float32)
        g = lax.dot_general(oht, t12[...].astype(jnp.bfloat16),
                            (((0,), (0,)), ((), ())),
                            preferred_element_type=jnp.float32)
        z = g + t + b_ref[...]
        h_ref[...] = z * jax.nn.sigmoid(z)

    return pl.pallas_call(
        body,
        grid=(nblk,),
        in_specs=[
            pl.BlockSpec((1, 1, Eb), lambda e: (e, 0, 0)),
            pl.BlockSpec((1, 1, Eb), lambda e: (e, 0, 0)),
            pl.BlockSpec((Eb, H), lambda e: (e + blk_off, 0)),
            pl.BlockSpec((H, H), lambda e: (0, 0)),
            pl.BlockSpec((H, 3 * H), lambda e: (0, 0)),
            pl.BlockSpec((1, H), lambda e: (0, 0)),
            pl.BlockSpec((1, H), lambda e: (0, 0)),
        ],
        out_specs=pl.BlockSpec((Eb, H), lambda e: (e, 0)),
        out_shape=jax.ShapeDtypeStruct((es, H), jnp.float32),
        scratch_shapes=[
            pltpu.VMEM((2 * H, H), jnp.float32),
            pltpu.VMEM((H, H), jnp.float32),
        ],
    )(xi, xj, rbf, embp, W, bias2, b2)


def _scatter_sc(h, rbf_out, i, zeros_nh, off):
    es, H = h.shape
    Npad = zeros_nh.shape[0]
    EW = es // NW
    CC = 40
    assert EW % CC == 0 and CC % 8 == 0
    assert Npad % (NS * 8) == 0
    RPT = Npad // NS
    mesh = plsc.VectorSubcoreMesh(core_axis_name="c", subcore_axis_name="s")

    @functools.partial(
        pl.kernel,
        mesh=mesh,
        out_type=(
            jax.ShapeDtypeStruct((Npad, H), jnp.float32),
            jax.ShapeDtypeStruct((Npad, H), jnp.float32),
        ),
        scratch_types=[
            pltpu.VMEM_SHARED((Npad, H), jnp.float32),
            pltpu.VMEM((CC, H), jnp.float32),
            pltpu.VMEM((CC, H), jnp.float32),
            pltpu.VMEM((CC, H), jnp.float32),
            pltpu.VMEM((CC, H), jnp.float32),
            pltpu.VMEM((CC,), jnp.int32),
            pltpu.VMEM((CC,), jnp.int32),
            pltpu.SemaphoreType.DMA,
            pltpu.SemaphoreType.DMA,
            pltpu.SemaphoreType.DMA,
            pltpu.SemaphoreType.DMA,
            pltpu.SemaphoreType.DMA,
            pltpu.SemaphoreType.DMA,
        ],
    )
    def k(h_hbm, ro_hbm, i_hbm, z_hbm, p0_hbm, p1_hbm, acc,
          hb0, hb1, rb0, rb1, ib0, ib1, hs0, hs1, rs0, rs1, is0, is1):
        c = lax.axis_index("c")
        s = lax.axis_index("s")
        w = s * NC + c
        rowsl = pl.ds(s * RPT, RPT)
        pltpu.sync_copy(z_hbm.at[rowsl], acc.at[rowsl])
        plsc.subcore_barrier()

        NCH = EW // CC
        slots = ((hb0, rb0, ib0, hs0, rs0, is0), (hb1, rb1, ib1, hs1, rs1, is1))

        def issue(m, slot):
            hb, rb, ib, hs, rs, isem = slot
            base = w * EW + m * CC
            pltpu.async_copy(h_hbm.at[pl.ds(base, CC)], hb, hs)
            pltpu.async_copy(ro_hbm.at[pl.ds(off + base, CC)], rb, rs)
            pltpu.async_copy(i_hbm.at[pl.ds(off + base, CC)], ib, isem)

        def drain(m, slot):
            hb, rb, ib, hs, rs, isem = slot
            base = w * EW + m * CC
            pltpu.make_async_copy(h_hbm.at[pl.ds(base, CC)], hb, hs).wait()
            pltpu.make_async_copy(ro_hbm.at[pl.ds(off + base, CC)], rb, rs).wait()
            pltpu.make_async_copy(i_hbm.at[pl.ds(off + base, CC)], ib, isem).wait()

        def process(m, slot, with_issue):
            hb, rb, ib = slot[0], slot[1], slot[2]
            drain(m, slot)

            @plsc.parallel_loop(0, CC, unroll=8)
            def _mul(r):
                for q in range(H // 16):
                    sl = pl.ds(q * 16, 16)
                    hb[r, sl] = hb[r, sl] * rb[r, sl]

            pltpu.sync_copy(hb, acc.at[ib], add=True)
            if with_issue:
                @pl.when(m <= NCH - 3)
                def _():
                    issue(m + 2, slot)

        issue(0, slots[0])
        issue(1, slots[1])

        def pair(g, _):
            process(2 * g, slots[0], True)
            process(2 * g + 1, slots[1], True)
            return 0

        lax.fori_loop(0, NCH // 2, pair, 0)
        if NCH % 2:
            process(NCH - 1, slots[0], False)
        plsc.subcore_barrier()

        @pl.when(c == 0)
        def _w0():
            pltpu.sync_copy(acc.at[rowsl], p0_hbm.at[rowsl])

        @pl.when(c == 1)
        def _w1():
            pltpu.sync_copy(acc.at[rowsl], p1_hbm.at[rowsl])

    return k(h, rbf_out, i, zeros_nh)


def _combine_tc(parts, N):
    H = parts[0].shape[1]
    Rb = 2000
    assert N % Rb == 0

    def body(a_ref, b_ref, c_ref, d_ref, o_ref):
        o_ref[...] = (a_ref[...] + b_ref[...]) + (c_ref[...] + d_ref[...])

    spec = pl.BlockSpec((Rb, H), lambda r: (r, 0))
    return pl.pallas_call(
        body,
        grid=(N // Rb,),
        in_specs=[spec] * 4,
        out_specs=spec,
        out_shape=jax.ShapeDtypeStruct((N, H), jnp.float32),
    )(*parts)


def kernel(x, rbf, i, j, rbf_out, num_nodes, emb, bias, W, b):
    del num_nodes
    N = x.shape[0]
    E = i.shape[0]
    H = rbf.shape[1]
    x = x.astype(jnp.int32)
    i = i.astype(jnp.int32)
    j = j.astype(jnp.int32)
    embp = jnp.zeros((H, H), jnp.float32).at[: emb.shape[0]].set(emb)
    bias2 = bias.reshape(1, H)
    b2 = b.reshape(1, H)
    npad = ((N + NS * 8 - 1) // (NS * 8)) * (NS * 8)
    zeros_nh = jnp.zeros((npad, H), jnp.float32)

    NSLICE = 2
    es = E // NSLICE
    Eb = 4000
    hs, parts = [], []
    for sidx in range(NSLICE):
        off = sidx * es
        xi, xj = _type_gather(x, i, j, off, es)
        h_s = _edge_tc(xi, xj, rbf, embp, W, bias2, b2,
                       blk_off=off // Eb, Eb=Eb)
        p0, p1 = _scatter_sc(h_s, rbf_out, i, zeros_nh, off)
        hs.append(h_s)
        parts += [p0, p1]
    h = jnp.concatenate(hs, axis=0)
    x_out = _combine_tc(parts, N)
    return (h, x_out)


# speedup vs baseline: 1.1048x; 1.1048x over previous
import functools

import jax
import jax.numpy as jnp
from jax import lax
from jax.experimental import pallas as pl
from jax.experimental.pallas import tpu as pltpu
from jax.experimental.pallas import tpu_sc as plsc

NC = 2
NS = 16
NW = NC * NS


def _type_gather(x, i, j):
    N = x.shape[0]
    E = i.shape[0]
    assert E % NW == 0
    EW = E // NW
    CA = 2000
    assert EW % CA == 0 and CA % 16 == 0
    mesh = plsc.VectorSubcoreMesh(core_axis_name="c", subcore_axis_name="s")

    @functools.partial(
        pl.kernel,
        mesh=mesh,
        out_type=(
            jax.ShapeDtypeStruct((E,), jnp.int32),
            jax.ShapeDtypeStruct((E,), jnp.int32),
        ),
        scratch_types=[
            pltpu.VMEM_SHARED((N,), jnp.int32),
            pltpu.VMEM((CA,), jnp.int32),
            pltpu.VMEM((CA,), jnp.int32),
            pltpu.VMEM((CA,), jnp.int32),
            pltpu.VMEM((CA,), jnp.int32),
            pltpu.SemaphoreType.DMA,
            pltpu.SemaphoreType.DMA,
            pltpu.SemaphoreType.DMA,
            pltpu.SemaphoreType.DMA,
            pltpu.SemaphoreType.DMA,
            pltpu.SemaphoreType.DMA,
        ],
    )
    def k(x_hbm, i_hbm, j_hbm, xi_hbm, xj_hbm,
          xsh, ib0, ib1, ob0, ob1, si0, si1, sg0, sg1, so0, so1):
        s_idx = lax.axis_index("s")
        w = s_idx * NC + lax.axis_index("c")

        @pl.when(s_idx == 0)
        def _stage():
            pltpu.sync_copy(x_hbm, xsh)

        plsc.subcore_barrier()
        nch = EW // CA
        units = [(i_hbm, xi_hbm, kk) for kk in range(nch)]
        units += [(j_hbm, xj_hbm, kk) for kk in range(nch)]
        slots = ((ib0, ob0, si0, sg0, so0), (ib1, ob1, si1, sg1, so1))
        nu = len(units)
        pend_in, pend_out = {}, {}

        def issue_in(m):
            src, _, kk = units[m]
            ib, _, si, _, _ = slots[m % 2]
            base = w * EW + kk * CA
            pend_in[m] = pltpu.async_copy(src.at[pl.ds(base, CA)], ib, si)

        issue_in(0)
        issue_in(1)
        for m in range(nu):
            ib, ob, si, sg, so = slots[m % 2]
            _, dst, kk = units[m]
            base = w * EW + kk * CA
            pend_in[m].wait()
            if m >= 2:
                pend_out[m - 2].wait()
            pltpu.async_copy(xsh.at[ib], ob, sg).wait()
            pend_out[m] = pltpu.async_copy(ob, dst.at[pl.ds(base, CA)], so)
            if m + 2 < nu:
                issue_in(m + 2)
        pend_out[nu - 2].wait()
        pend_out[nu - 1].wait()

    return k(x, i, j)


def _edge_tc(xi, xj, rbf, embp, W, bias2, b2):
    E, H = rbf.shape
    Eb = 4000
    assert E % Eb == 0
    xi = xi.reshape(E // Eb, 1, Eb)
    xj = xj.reshape(E // Eb, 1, Eb)

    def body(xi_ref, xj_ref, rbf_ref, emb_ref, w_ref, bias_ref, b_ref,
             h_ref, t12, w3):
        @pl.when(pl.program_id(0) == 0)
        def _init():
            embv = emb_ref[...]
            dn = (((1,), (1,)), ((), ()))
            t12[0:H, :] = lax.dot_general(
                embv, w_ref[:, 0:H], dn, preferred_element_type=jnp.float32)
            t12[H:2 * H, :] = lax.dot_general(
                embv, w_ref[:, H:2 * H], dn, preferred_element_type=jnp.float32)
            w3[...] = w_ref[:, 2 * H:3 * H]

        xi_v = xi_ref[...].reshape(1, Eb)
        xj_v = xj_ref[...].reshape(1, Eb) + H
        iot = lax.broadcasted_iota(jnp.int32, (2 * H, Eb), 0)
        oht = jnp.where((iot == xi_v) | (iot == xj_v),
                        1.0, 0.0).astype(jnp.bfloat16)
        ra = rbf_ref[...] + bias_ref[...]
        ra = ra * jax.nn.sigmoid(ra)
        t = lax.dot_general(ra.astype(jnp.bfloat16),
                            w3[...].astype(jnp.bfloat16),
                            (((1,), (1,)), ((), ())),
                            preferred_element_type=jnp.float32)
        g = lax.dot_general(oht, t12[...].astype(jnp.bfloat16),
                            (((0,), (0,)), ((), ())),
                            preferred_element_type=jnp.float32)
        z = g + t + b_ref[...]
        h_ref[...] = z * jax.nn.sigmoid(z)

    return pl.pallas_call(
        body,
        grid=(E // Eb,),
        in_specs=[
            pl.BlockSpec((1, 1, Eb), lambda e: (e, 0, 0)),
            pl.BlockSpec((1, 1, Eb), lambda e: (e, 0, 0)),
            pl.BlockSpec((Eb, H), lambda e: (e, 0)),
            pl.BlockSpec((H, H), lambda e: (0, 0)),
            pl.BlockSpec((H, 3 * H), lambda e: (0, 0)),
            pl.BlockSpec((1, H), lambda e: (0, 0)),
            pl.BlockSpec((1, H), lambda e: (0, 0)),
        ],
        out_specs=pl.BlockSpec((Eb, H), lambda e: (e, 0)),
        out_shape=jax.ShapeDtypeStruct((E, H), jnp.float32),
        scratch_shapes=[
            pltpu.VMEM((2 * H, H), jnp.float32),
            pltpu.VMEM((H, H), jnp.float32),
        ],
    )(xi, xj, rbf, embp, W, bias2, b2)


def _scatter_sc(h, rbf_out, i, zeros_nh):
    E, H = h.shape
    Npad = zeros_nh.shape[0]
    EW = E // NW
    CC = 80
    assert EW % CC == 0 and CC % 8 == 0
    assert Npad % (NS * 8) == 0
    RPT = Npad // NS
    mesh = plsc.VectorSubcoreMesh(core_axis_name="c", subcore_axis_name="s")

    @functools.partial(
        pl.kernel,
        mesh=mesh,
        out_type=(
            jax.ShapeDtypeStruct((Npad, H), jnp.float32),
            jax.ShapeDtypeStruct((Npad, H), jnp.float32),
        ),
        scratch_types=[
            pltpu.VMEM_SHARED((Npad, H), jnp.float32),
            pltpu.VMEM((CC, H), jnp.float32),
            pltpu.VMEM((CC, H), jnp.float32),
            pltpu.VMEM((CC, H), jnp.float32),
            pltpu.VMEM((CC, H), jnp.float32),
            pltpu.VMEM((CC,), jnp.int32),
            pltpu.VMEM((CC,), jnp.int32),
            pltpu.SemaphoreType.DMA,
            pltpu.SemaphoreType.DMA,
            pltpu.SemaphoreType.DMA,
            pltpu.SemaphoreType.DMA,
            pltpu.SemaphoreType.DMA,
            pltpu.SemaphoreType.DMA,
        ],
    )
    def k(h_hbm, ro_hbm, i_hbm, z_hbm, p0_hbm, p1_hbm, acc,
          hb0, hb1, rb0, rb1, ib0, ib1, hs0, hs1, rs0, rs1, is0, is1):
        c = lax.axis_index("c")
        s = lax.axis_index("s")
        w = s * NC + c
        rowsl = pl.ds(s * RPT, RPT)
        pltpu.sync_copy(z_hbm.at[rowsl], acc.at[rowsl])
        plsc.subcore_barrier()

        NCH = EW // CC
        slots = ((hb0, rb0, ib0, hs0, rs0, is0), (hb1, rb1, ib1, hs1, rs1, is1))

        def issue(m, slot):
            hb, rb, ib, hs, rs, isem = slot
            base = w * EW + m * CC
            pltpu.async_copy(h_hbm.at[pl.ds(base, CC)], hb, hs)
            pltpu.async_copy(ro_hbm.at[pl.ds(base, CC)], rb, rs)
            pltpu.async_copy(i_hbm.at[pl.ds(base, CC)], ib, isem)

        def drain(m, slot):
            hb, rb, ib, hs, rs, isem = slot
            base = w * EW + m * CC
            pltpu.make_async_copy(h_hbm.at[pl.ds(base, CC)], hb, hs).wait()
            pltpu.make_async_copy(ro_hbm.at[pl.ds(base, CC)], rb, rs).wait()
            pltpu.make_async_copy(i_hbm.at[pl.ds(base, CC)], ib, isem).wait()

        def process(m, slot, with_issue):
            hb, rb, ib = slot[0], slot[1], slot[2]
            drain(m, slot)

            @plsc.parallel_loop(0, CC, unroll=8)
            def _mul(r):
                for q in range(H // 16):
                    sl = pl.ds(q * 16, 16)
                    hb[r, sl] = hb[r, sl] * rb[r, sl]

            pltpu.sync_copy(hb, acc.at[ib], add=True)
            if with_issue:
                @pl.when(m <= NCH - 3)
                def _():
                    issue(m + 2, slot)

        issue(0, slots[0])
        issue(1, slots[1])

        def pair(g, _):
            process(2 * g, slots[0], True)
            process(2 * g + 1, slots[1], True)
            return 0

        lax.fori_loop(0, NCH // 2, pair, 0)
        if NCH % 2:
            process(NCH - 1, slots[0], False)
        plsc.subcore_barrier()

        @pl.when(c == 0)
        def _w0():
            pltpu.sync_copy(acc.at[rowsl], p0_hbm.at[rowsl])

        @pl.when(c == 1)
        def _w1():
            pltpu.sync_copy(acc.at[rowsl], p1_hbm.at[rowsl])

    return k(h, rbf_out, i, zeros_nh)


def _combine_tc(p0, p1, N):
    H = p0.shape[1]
    Rb = 2000
    assert N % Rb == 0

    def body(a_ref, b_ref, o_ref):
        o_ref[...] = a_ref[...] + b_ref[...]

    return pl.pallas_call(
        body,
        grid=(N // Rb,),
        in_specs=[
            pl.BlockSpec((Rb, H), lambda r: (r, 0)),
            pl.BlockSpec((Rb, H), lambda r: (r, 0)),
        ],
        out_specs=pl.BlockSpec((Rb, H), lambda r: (r, 0)),
        out_shape=jax.ShapeDtypeStruct((N, H), jnp.float32),
    )(p0, p1)


def kernel(x, rbf, i, j, rbf_out, num_nodes, emb, bias, W, b):
    del num_nodes
    N = x.shape[0]
    E = i.shape[0]
    H = rbf.shape[1]
    x = x.astype(jnp.int32)
    i = i.astype(jnp.int32)
    j = j.astype(jnp.int32)
    xi, xj = _type_gather(x, i, j)
    embp = jnp.zeros((H, H), jnp.float32).at[: emb.shape[0]].set(emb)
    h = _edge_tc(xi, xj, rbf, embp, W,
                 bias.reshape(1, H), b.reshape(1, H))
    npad = ((N + NS * 8 - 1) // (NS * 8)) * (NS * 8)
    zeros_nh = jnp.zeros((npad, H), jnp.float32)
    p0, p1 = _scatter_sc(h, rbf_out, i, zeros_nh)
    x_out = _combine_tc(p0, p1, N)
    return (h, x_out)


# speedup vs baseline: 1.1694x; 1.0585x over previous
import functools

import jax
import jax.numpy as jnp
from jax import lax
from jax.experimental import pallas as pl
from jax.experimental.pallas import tpu as pltpu
from jax.experimental.pallas import tpu_sc as plsc

NC = 2
NS = 16
NW = NC * NS


def _type_gather(x, i, j):
    N = x.shape[0]
    E = i.shape[0]
    assert E % NW == 0
    EW = E // NW
    CA = 2000
    assert EW % CA == 0 and CA % 16 == 0
    mesh = plsc.VectorSubcoreMesh(core_axis_name="c", subcore_axis_name="s")

    @functools.partial(
        pl.kernel,
        mesh=mesh,
        out_type=(
            jax.ShapeDtypeStruct((E,), jnp.int32),
            jax.ShapeDtypeStruct((E,), jnp.int32),
        ),
        scratch_types=[
            pltpu.VMEM_SHARED((N,), jnp.int32),
            pltpu.VMEM((CA,), jnp.int32),
            pltpu.VMEM((CA,), jnp.int32),
            pltpu.VMEM((CA,), jnp.int32),
            pltpu.VMEM((CA,), jnp.int32),
            pltpu.SemaphoreType.DMA,
            pltpu.SemaphoreType.DMA,
            pltpu.SemaphoreType.DMA,
            pltpu.SemaphoreType.DMA,
            pltpu.SemaphoreType.DMA,
            pltpu.SemaphoreType.DMA,
        ],
    )
    def k(x_hbm, i_hbm, j_hbm, xi_hbm, xj_hbm,
          xsh, ib0, ib1, ob0, ob1, si0, si1, sg0, sg1, so0, so1):
        s_idx = lax.axis_index("s")
        w = s_idx * NC + lax.axis_index("c")

        @pl.when(s_idx == 0)
        def _stage():
            pltpu.sync_copy(x_hbm, xsh)

        plsc.subcore_barrier()
        nch = EW // CA
        units = [(i_hbm, xi_hbm, kk) for kk in range(nch)]
        units += [(j_hbm, xj_hbm, kk) for kk in range(nch)]
        slots = ((ib0, ob0, si0, sg0, so0), (ib1, ob1, si1, sg1, so1))
        nu = len(units)
        pend_in, pend_out = {}, {}

        def issue_in(m):
            src, _, kk = units[m]
            ib, _, si, _, _ = slots[m % 2]
            base = w * EW + kk * CA
            pend_in[m] = pltpu.async_copy(src.at[pl.ds(base, CA)], ib, si)

        issue_in(0)
        issue_in(1)
        for m in range(nu):
            ib, ob, si, sg, so = slots[m % 2]
            _, dst, kk = units[m]
            base = w * EW + kk * CA
            pend_in[m].wait()
            if m >= 2:
                pend_out[m - 2].wait()
            pltpu.async_copy(xsh.at[ib], ob, sg).wait()
            pend_out[m] = pltpu.async_copy(ob, dst.at[pl.ds(base, CA)], so)
            if m + 2 < nu:
                issue_in(m + 2)
        pend_out[nu - 2].wait()
        pend_out[nu - 1].wait()

    return k(x, i, j)


def _edge_tc(xi, xj, rbf, embp, W, bias2, b2):
    E, H = rbf.shape
    Eb = 8000
    assert E % Eb == 0
    xi = xi.reshape(E // Eb, 1, Eb)
    xj = xj.reshape(E // Eb, 1, Eb)

    def body(xi_ref, xj_ref, rbf_ref, emb_ref, w_ref, bias_ref, b_ref,
             h_ref, t12, w3):
        @pl.when(pl.program_id(0) == 0)
        def _init():
            embv = emb_ref[...]
            dn = (((1,), (1,)), ((), ()))
            t12[0:H, :] = lax.dot_general(
                embv, w_ref[:, 0:H], dn, preferred_element_type=jnp.float32)
            t12[H:2 * H, :] = lax.dot_general(
                embv, w_ref[:, H:2 * H], dn, preferred_element_type=jnp.float32)
            w3[...] = w_ref[:, 2 * H:3 * H]

        xi_v = xi_ref[...].reshape(1, Eb)
        xj_v = xj_ref[...].reshape(1, Eb) + H
        iot = lax.broadcasted_iota(jnp.int32, (2 * H, Eb), 0)
        oht = jnp.where((iot == xi_v) | (iot == xj_v),
                        1.0, 0.0).astype(jnp.bfloat16)
        ra = rbf_ref[...] + bias_ref[...]
        ra = ra * jax.nn.sigmoid(ra)
        t = lax.dot_general(ra.astype(jnp.bfloat16),
                            w3[...].astype(jnp.bfloat16),
                            (((1,), (1,)), ((), ())),
                            preferred_element_type=jnp.float32)
        g = lax.dot_general(oht, t12[...].astype(jnp.bfloat16),
                            (((0,), (0,)), ((), ())),
                            preferred_element_type=jnp.float32)
        z = g + t + b_ref[...]
        h_ref[...] = z * jax.nn.sigmoid(z)

    return pl.pallas_call(
        body,
        grid=(E // Eb,),
        in_specs=[
            pl.BlockSpec((1, 1, Eb), lambda e: (e, 0, 0)),
            pl.BlockSpec((1, 1, Eb), lambda e: (e, 0, 0)),
            pl.BlockSpec((Eb, H), lambda e: (e, 0)),
            pl.BlockSpec((H, H), lambda e: (0, 0)),
            pl.BlockSpec((H, 3 * H), lambda e: (0, 0)),
            pl.BlockSpec((1, H), lambda e: (0, 0)),
            pl.BlockSpec((1, H), lambda e: (0, 0)),
        ],
        out_specs=pl.BlockSpec((Eb, H), lambda e: (e, 0)),
        out_shape=jax.ShapeDtypeStruct((E, H), jnp.float32),
        scratch_shapes=[
            pltpu.VMEM((2 * H, H), jnp.float32),
            pltpu.VMEM((H, H), jnp.float32),
        ],
    )(xi, xj, rbf, embp, W, bias2, b2)


def _scatter_sc(h, rbf_out, i, zeros_nh):
    E, H = h.shape
    Npad = zeros_nh.shape[0]
    EW = E // NW
    CC = 80
    assert EW % CC == 0 and CC % 8 == 0
    assert Npad % (NS * 8) == 0
    RPT = Npad // NS
    mesh = plsc.VectorSubcoreMesh(core_axis_name="c", subcore_axis_name="s")

    @functools.partial(
        pl.kernel,
        mesh=mesh,
        out_type=(
            jax.ShapeDtypeStruct((Npad, H), jnp.float32),
            jax.ShapeDtypeStruct((Npad, H), jnp.float32),
        ),
        scratch_types=[
            pltpu.VMEM_SHARED((Npad, H), jnp.float32),
            pltpu.VMEM((CC, H), jnp.float32),
            pltpu.VMEM((CC, H), jnp.float32),
            pltpu.VMEM((CC, H), jnp.float32),
            pltpu.VMEM((CC, H), jnp.float32),
            pltpu.VMEM((CC,), jnp.int32),
            pltpu.VMEM((CC,), jnp.int32),
            pltpu.SemaphoreType.DMA,
            pltpu.SemaphoreType.DMA,
            pltpu.SemaphoreType.DMA,
            pltpu.SemaphoreType.DMA,
            pltpu.SemaphoreType.DMA,
            pltpu.SemaphoreType.DMA,
        ],
    )
    def k(h_hbm, ro_hbm, i_hbm, z_hbm, p0_hbm, p1_hbm, acc,
          hb0, hb1, rb0, rb1, ib0, ib1, hs0, hs1, rs0, rs1, is0, is1):
        c = lax.axis_index("c")
        s = lax.axis_index("s")
        w = s * NC + c
        rowsl = pl.ds(s * RPT, RPT)
        pltpu.sync_copy(z_hbm.at[rowsl], acc.at[rowsl])
        plsc.subcore_barrier()

        NCH = EW // CC
        slots = ((hb0, rb0, ib0, hs0, rs0, is0), (hb1, rb1, ib1, hs1, rs1, is1))

        def issue(m, slot):
            hb, rb, ib, hs, rs, isem = slot
            base = w * EW + m * CC
            pltpu.async_copy(h_hbm.at[pl.ds(base, CC)], hb, hs)
            pltpu.async_copy(ro_hbm.at[pl.ds(base, CC)], rb, rs)
            pltpu.async_copy(i_hbm.at[pl.ds(base, CC)], ib, isem)

        def drain(m, slot):
            hb, rb, ib, hs, rs, isem = slot
            base = w * EW + m * CC
            pltpu.make_async_copy(h_hbm.at[pl.ds(base, CC)], hb, hs).wait()
            pltpu.make_async_copy(ro_hbm.at[pl.ds(base, CC)], rb, rs).wait()
            pltpu.make_async_copy(i_hbm.at[pl.ds(base, CC)], ib, isem).wait()

        def process(m, slot, with_issue):
            hb, rb, ib = slot[0], slot[1], slot[2]
            drain(m, slot)

            @plsc.parallel_loop(0, CC, unroll=8)
            def _mul(r):
                for q in range(H // 16):
                    sl = pl.ds(q * 16, 16)
                    hb[r, sl] = hb[r, sl] * rb[r, sl]

            pltpu.sync_copy(hb, acc.at[ib], add=True)
            if with_issue:
                @pl.when(m <= NCH - 3)
                def _():
                    issue(m + 2, slot)

        issue(0, slots[0])
        issue(1, slots[1])

        def pair(g, _):
            process(2 * g, slots[0], True)
            process(2 * g + 1, slots[1], True)
            return 0

        lax.fori_loop(0, NCH // 2, pair, 0)
        if NCH % 2:
            process(NCH - 1, slots[0], False)
        plsc.subcore_barrier()

        @pl.when(c == 0)
        def _w0():
            pltpu.sync_copy(acc.at[rowsl], p0_hbm.at[rowsl])

        @pl.when(c == 1)
        def _w1():
            pltpu.sync_copy(acc.at[rowsl], p1_hbm.at[rowsl])

    return k(h, rbf_out, i, zeros_nh)


def _combine_tc(p0, p1, N):
    H = p0.shape[1]
    Rb = 2000
    assert N % Rb == 0

    def body(a_ref, b_ref, o_ref):
        o_ref[...] = a_ref[...] + b_ref[...]

    return pl.pallas_call(
        body,
        grid=(N // Rb,),
        in_specs=[
            pl.BlockSpec((Rb, H), lambda r: (r, 0)),
            pl.BlockSpec((Rb, H), lambda r: (r, 0)),
        ],
        out_specs=pl.BlockSpec((Rb, H), lambda r: (r, 0)),
        out_shape=jax.ShapeDtypeStruct((N, H), jnp.float32),
    )(p0, p1)


def kernel(x, rbf, i, j, rbf_out, num_nodes, emb, bias, W, b):
    del num_nodes
    N = x.shape[0]
    E = i.shape[0]
    H = rbf.shape[1]
    x = x.astype(jnp.int32)
    i = i.astype(jnp.int32)
    j = j.astype(jnp.int32)
    xi, xj = _type_gather(x, i, j)
    embp = jnp.zeros((H, H), jnp.float32).at[: emb.shape[0]].set(emb)
    h = _edge_tc(xi, xj, rbf, embp, W,
                 bias.reshape(1, H), b.reshape(1, H))
    npad = ((N + NS * 8 - 1) // (NS * 8)) * (NS * 8)
    zeros_nh = jnp.zeros((npad, H), jnp.float32)
    p0, p1 = _scatter_sc(h, rbf_out, i, zeros_nh)
    x_out = _combine_tc(p0, p1, N)
    return (h, x_out)


# speedup vs baseline: 1.2023x; 1.0281x over previous
import functools

import jax
import jax.numpy as jnp
from jax import lax
from jax.experimental import pallas as pl
from jax.experimental.pallas import tpu as pltpu
from jax.experimental.pallas import tpu_sc as plsc

NC = 2
NS = 16
NW = NC * NS


def _type_gather(x, i, j):
    N = x.shape[0]
    E = i.shape[0]
    assert E % NW == 0
    EW = E // NW
    CA = 2000
    assert EW % CA == 0 and CA % 16 == 0
    mesh = plsc.VectorSubcoreMesh(core_axis_name="c", subcore_axis_name="s")

    @functools.partial(
        pl.kernel,
        mesh=mesh,
        out_type=(
            jax.ShapeDtypeStruct((E,), jnp.int32),
            jax.ShapeDtypeStruct((E,), jnp.int32),
        ),
        scratch_types=[
            pltpu.VMEM_SHARED((N,), jnp.int32),
            pltpu.VMEM((CA,), jnp.int32),
            pltpu.VMEM((CA,), jnp.int32),
            pltpu.VMEM((CA,), jnp.int32),
            pltpu.VMEM((CA,), jnp.int32),
            pltpu.SemaphoreType.DMA,
            pltpu.SemaphoreType.DMA,
            pltpu.SemaphoreType.DMA,
            pltpu.SemaphoreType.DMA,
            pltpu.SemaphoreType.DMA,
            pltpu.SemaphoreType.DMA,
        ],
    )
    def k(x_hbm, i_hbm, j_hbm, xi_hbm, xj_hbm,
          xsh, ib0, ib1, ob0, ob1, si0, si1, sg0, sg1, so0, so1):
        s_idx = lax.axis_index("s")
        w = s_idx * NC + lax.axis_index("c")

        @pl.when(s_idx == 0)
        def _stage():
            pltpu.sync_copy(x_hbm, xsh)

        plsc.subcore_barrier()
        nch = EW // CA
        units = [(i_hbm, xi_hbm, kk) for kk in range(nch)]
        units += [(j_hbm, xj_hbm, kk) for kk in range(nch)]
        slots = ((ib0, ob0, si0, sg0, so0), (ib1, ob1, si1, sg1, so1))
        nu = len(units)
        pend_in, pend_out = {}, {}

        def issue_in(m):
            src, _, kk = units[m]
            ib, _, si, _, _ = slots[m % 2]
            base = w * EW + kk * CA
            pend_in[m] = pltpu.async_copy(src.at[pl.ds(base, CA)], ib, si)

        issue_in(0)
        issue_in(1)
        for m in range(nu):
            ib, ob, si, sg, so = slots[m % 2]
            _, dst, kk = units[m]
            base = w * EW + kk * CA
            pend_in[m].wait()
            if m >= 2:
                pend_out[m - 2].wait()
            pltpu.async_copy(xsh.at[ib], ob, sg).wait()
            pend_out[m] = pltpu.async_copy(ob, dst.at[pl.ds(base, CA)], so)
            if m + 2 < nu:
                issue_in(m + 2)
        pend_out[nu - 2].wait()
        pend_out[nu - 1].wait()

    return k(x, i, j)


def _edge_tc(xi, xj, rbf, embp, W, bias2, b2):
    E, H = rbf.shape
    Eb = 16000
    assert E % Eb == 0
    xi = xi.reshape(E // Eb, 1, Eb)
    xj = xj.reshape(E // Eb, 1, Eb)

    def body(xi_ref, xj_ref, rbf_ref, emb_ref, w_ref, bias_ref, b_ref,
             h_ref, t12, w3):
        @pl.when(pl.program_id(0) == 0)
        def _init():
            embv = emb_ref[...]
            dn = (((1,), (1,)), ((), ()))
            t12[0:H, :] = lax.dot_general(
                embv, w_ref[:, 0:H], dn, preferred_element_type=jnp.float32)
            t12[H:2 * H, :] = lax.dot_general(
                embv, w_ref[:, H:2 * H], dn, preferred_element_type=jnp.float32)
            w3[...] = w_ref[:, 2 * H:3 * H]

        xi_v = xi_ref[...].reshape(1, Eb)
        xj_v = xj_ref[...].reshape(1, Eb) + H
        iot = lax.broadcasted_iota(jnp.int32, (2 * H, Eb), 0)
        oht = jnp.where((iot == xi_v) | (iot == xj_v),
                        1.0, 0.0).astype(jnp.bfloat16)
        ra = rbf_ref[...] + bias_ref[...]
        ra = ra * jax.nn.sigmoid(ra)
        t = lax.dot_general(ra.astype(jnp.bfloat16),
                            w3[...].astype(jnp.bfloat16),
                            (((1,), (1,)), ((), ())),
                            preferred_element_type=jnp.float32)
        g = lax.dot_general(oht, t12[...].astype(jnp.bfloat16),
                            (((0,), (0,)), ((), ())),
                            preferred_element_type=jnp.float32)
        z = g + t + b_ref[...]
        h_ref[...] = z * jax.nn.sigmoid(z)

    return pl.pallas_call(
        body,
        grid=(E // Eb,),
        in_specs=[
            pl.BlockSpec((1, 1, Eb), lambda e: (e, 0, 0)),
            pl.BlockSpec((1, 1, Eb), lambda e: (e, 0, 0)),
            pl.BlockSpec((Eb, H), lambda e: (e, 0)),
            pl.BlockSpec((H, H), lambda e: (0, 0)),
            pl.BlockSpec((H, 3 * H), lambda e: (0, 0)),
            pl.BlockSpec((1, H), lambda e: (0, 0)),
            pl.BlockSpec((1, H), lambda e: (0, 0)),
        ],
        out_specs=pl.BlockSpec((Eb, H), lambda e: (e, 0)),
        out_shape=jax.ShapeDtypeStruct((E, H), jnp.float32),
        scratch_shapes=[
            pltpu.VMEM((2 * H, H), jnp.float32),
            pltpu.VMEM((H, H), jnp.float32),
        ],
    )(xi, xj, rbf, embp, W, bias2, b2)


def _scatter_sc(h, rbf_out, i, zeros_nh):
    E, H = h.shape
    Npad = zeros_nh.shape[0]
    EW = E // NW
    CC = 80
    assert EW % CC == 0 and CC % 8 == 0
    assert Npad % (NS * 8) == 0
    RPT = Npad // NS
    mesh = plsc.VectorSubcoreMesh(core_axis_name="c", subcore_axis_name="s")

    @functools.partial(
        pl.kernel,
        mesh=mesh,
        out_type=(
            jax.ShapeDtypeStruct((Npad, H), jnp.float32),
            jax.ShapeDtypeStruct((Npad, H), jnp.float32),
        ),
        scratch_types=[
            pltpu.VMEM_SHARED((Npad, H), jnp.float32),
            pltpu.VMEM((CC, H), jnp.float32),
            pltpu.VMEM((CC, H), jnp.float32),
            pltpu.VMEM((CC, H), jnp.float32),
            pltpu.VMEM((CC, H), jnp.float32),
            pltpu.VMEM((CC,), jnp.int32),
            pltpu.VMEM((CC,), jnp.int32),
            pltpu.SemaphoreType.DMA,
            pltpu.SemaphoreType.DMA,
            pltpu.SemaphoreType.DMA,
            pltpu.SemaphoreType.DMA,
            pltpu.SemaphoreType.DMA,
            pltpu.SemaphoreType.DMA,
        ],
    )
    def k(h_hbm, ro_hbm, i_hbm, z_hbm, p0_hbm, p1_hbm, acc,
          hb0, hb1, rb0, rb1, ib0, ib1, hs0, hs1, rs0, rs1, is0, is1):
        c = lax.axis_index("c")
        s = lax.axis_index("s")
        w = s * NC + c
        rowsl = pl.ds(s * RPT, RPT)
        pltpu.sync_copy(z_hbm.at[rowsl], acc.at[rowsl])
        plsc.subcore_barrier()

        NCH = EW // CC
        slots = ((hb0, rb0, ib0, hs0, rs0, is0), (hb1, rb1, ib1, hs1, rs1, is1))

        def issue(m, slot):
            hb, rb, ib, hs, rs, isem = slot
            base = w * EW + m * CC
            pltpu.async_copy(h_hbm.at[pl.ds(base, CC)], hb, hs)
            pltpu.async_copy(ro_hbm.at[pl.ds(base, CC)], rb, rs)
            pltpu.async_copy(i_hbm.at[pl.ds(base, CC)], ib, isem)

        def drain(m, slot):
            hb, rb, ib, hs, rs, isem = slot
            base = w * EW + m * CC
            pltpu.make_async_copy(h_hbm.at[pl.ds(base, CC)], hb, hs).wait()
            pltpu.make_async_copy(ro_hbm.at[pl.ds(base, CC)], rb, rs).wait()
            pltpu.make_async_copy(i_hbm.at[pl.ds(base, CC)], ib, isem).wait()

        def process(m, slot, with_issue):
            hb, rb, ib = slot[0], slot[1], slot[2]
            drain(m, slot)

            @plsc.parallel_loop(0, CC, unroll=8)
            def _mul(r):
                for q in range(H // 16):
                    sl = pl.ds(q * 16, 16)
                    hb[r, sl] = hb[r, sl] * rb[r, sl]

            pltpu.sync_copy(hb, acc.at[ib], add=True)
            if with_issue:
                @pl.when(m <= NCH - 3)
                def _():
                    issue(m + 2, slot)

        issue(0, slots[0])
        issue(1, slots[1])

        def pair(g, _):
            process(2 * g, slots[0], True)
            process(2 * g + 1, slots[1], True)
            return 0

        lax.fori_loop(0, NCH // 2, pair, 0)
        if NCH % 2:
            process(NCH - 1, slots[0], False)
        plsc.subcore_barrier()

        @pl.when(c == 0)
        def _w0():
            pltpu.sync_copy(acc.at[rowsl], p0_hbm.at[rowsl])

        @pl.when(c == 1)
        def _w1():
            pltpu.sync_copy(acc.at[rowsl], p1_hbm.at[rowsl])

    return k(h, rbf_out, i, zeros_nh)


def _combine_tc(p0, p1, N):
    H = p0.shape[1]
    Rb = 2000
    assert N % Rb == 0

    def body(a_ref, b_ref, o_ref):
        o_ref[...] = a_ref[...] + b_ref[...]

    return pl.pallas_call(
        body,
        grid=(N // Rb,),
        in_specs=[
            pl.BlockSpec((Rb, H), lambda r: (r, 0)),
            pl.BlockSpec((Rb, H), lambda r: (r, 0)),
        ],
        out_specs=pl.BlockSpec((Rb, H), lambda r: (r, 0)),
        out_shape=jax.ShapeDtypeStruct((N, H), jnp.float32),
    )(p0, p1)


def kernel(x, rbf, i, j, rbf_out, num_nodes, emb, bias, W, b):
    del num_nodes
    N = x.shape[0]
    E = i.shape[0]
    H = rbf.shape[1]
    x = x.astype(jnp.int32)
    i = i.astype(jnp.int32)
    j = j.astype(jnp.int32)
    xi, xj = _type_gather(x, i, j)
    embp = jnp.zeros((H, H), jnp.float32).at[: emb.shape[0]].set(emb)
    h = _edge_tc(xi, xj, rbf, embp, W,
                 bias.reshape(1, H), b.reshape(1, H))
    npad = ((N + NS * 8 - 1) // (NS * 8)) * (NS * 8)
    zeros_nh = jnp.zeros((npad, H), jnp.float32)
    p0, p1 = _scatter_sc(h, rbf_out, i, zeros_nh)
    x_out = _combine_tc(p0, p1, N)
    return (h, x_out)


# speedup vs baseline: 1.2074x; 1.0042x over previous
import functools

import jax
import jax.numpy as jnp
from jax import lax
from jax.experimental import pallas as pl
from jax.experimental.pallas import tpu as pltpu
from jax.experimental.pallas import tpu_sc as plsc

NC = 2
NS = 16
NW = NC * NS


def _type_gather(x, i, j):
    N = x.shape[0]
    E = i.shape[0]
    assert E % NW == 0
    EW = E // NW
    CA = 2000
    assert EW % CA == 0 and CA % 16 == 0
    mesh = plsc.VectorSubcoreMesh(core_axis_name="c", subcore_axis_name="s")

    @functools.partial(
        pl.kernel,
        mesh=mesh,
        out_type=(
            jax.ShapeDtypeStruct((E,), jnp.int32),
            jax.ShapeDtypeStruct((E,), jnp.int32),
        ),
        scratch_types=[
            pltpu.VMEM_SHARED((N,), jnp.int32),
            pltpu.VMEM((CA,), jnp.int32),
            pltpu.VMEM((CA,), jnp.int32),
            pltpu.VMEM((CA,), jnp.int32),
            pltpu.VMEM((CA,), jnp.int32),
            pltpu.SemaphoreType.DMA,
            pltpu.SemaphoreType.DMA,
            pltpu.SemaphoreType.DMA,
            pltpu.SemaphoreType.DMA,
            pltpu.SemaphoreType.DMA,
            pltpu.SemaphoreType.DMA,
        ],
    )
    def k(x_hbm, i_hbm, j_hbm, xi_hbm, xj_hbm,
          xsh, ib0, ib1, ob0, ob1, si0, si1, sg0, sg1, so0, so1):
        s_idx = lax.axis_index("s")
        w = s_idx * NC + lax.axis_index("c")

        @pl.when(s_idx == 0)
        def _stage():
            pltpu.sync_copy(x_hbm, xsh)

        plsc.subcore_barrier()
        nch = EW // CA
        units = [(i_hbm, xi_hbm, kk) for kk in range(nch)]
        units += [(j_hbm, xj_hbm, kk) for kk in range(nch)]
        slots = ((ib0, ob0, si0, sg0, so0), (ib1, ob1, si1, sg1, so1))
        nu = len(units)
        pend_in, pend_out = {}, {}

        def issue_in(m):
            src, _, kk = units[m]
            ib, _, si, _, _ = slots[m % 2]
            base = w * EW + kk * CA
            pend_in[m] = pltpu.async_copy(src.at[pl.ds(base, CA)], ib, si)

        issue_in(0)
        issue_in(1)
        for m in range(nu):
            ib, ob, si, sg, so = slots[m % 2]
            _, dst, kk = units[m]
            base = w * EW + kk * CA
            pend_in[m].wait()
            if m >= 2:
                pend_out[m - 2].wait()
            pltpu.async_copy(xsh.at[ib], ob, sg).wait()
            pend_out[m] = pltpu.async_copy(ob, dst.at[pl.ds(base, CA)], so)
            if m + 2 < nu:
                issue_in(m + 2)
        pend_out[nu - 2].wait()
        pend_out[nu - 1].wait()

    return k(x, i, j)


def _edge_tc(xi, xj, rbf, embp, W, bias2, b2):
    E, H = rbf.shape
    Eb = 20000
    assert E % Eb == 0
    xi = xi.reshape(E // Eb, 1, Eb)
    xj = xj.reshape(E // Eb, 1, Eb)

    def body(xi_ref, xj_ref, rbf_ref, emb_ref, w_ref, bias_ref, b_ref,
             h_ref, t12, w3):
        @pl.when(pl.program_id(0) == 0)
        def _init():
            embv = emb_ref[...]
            dn = (((1,), (1,)), ((), ()))
            t12[0:H, :] = lax.dot_general(
                embv, w_ref[:, 0:H], dn, preferred_element_type=jnp.float32)
            t12[H:2 * H, :] = lax.dot_general(
                embv, w_ref[:, H:2 * H], dn, preferred_element_type=jnp.float32)
            w3[...] = w_ref[:, 2 * H:3 * H]

        xi_v = xi_ref[...].reshape(1, Eb)
        xj_v = xj_ref[...].reshape(1, Eb) + H
        iot = lax.broadcasted_iota(jnp.int32, (2 * H, Eb), 0)
        oht = jnp.where((iot == xi_v) | (iot == xj_v),
                        1.0, 0.0).astype(jnp.bfloat16)
        ra = rbf_ref[...] + bias_ref[...]
        ra = ra * jax.nn.sigmoid(ra)
        t = lax.dot_general(ra.astype(jnp.bfloat16),
                            w3[...].astype(jnp.bfloat16),
                            (((1,), (1,)), ((), ())),
                            preferred_element_type=jnp.float32)
        g = lax.dot_general(oht, t12[...].astype(jnp.bfloat16),
                            (((0,), (0,)), ((), ())),
                            preferred_element_type=jnp.float32)
        z = g + t + b_ref[...]
        h_ref[...] = z * jax.nn.sigmoid(z)

    return pl.pallas_call(
        body,
        grid=(E // Eb,),
        in_specs=[
            pl.BlockSpec((1, 1, Eb), lambda e: (e, 0, 0)),
            pl.BlockSpec((1, 1, Eb), lambda e: (e, 0, 0)),
            pl.BlockSpec((Eb, H), lambda e: (e, 0)),
            pl.BlockSpec((H, H), lambda e: (0, 0)),
            pl.BlockSpec((H, 3 * H), lambda e: (0, 0)),
            pl.BlockSpec((1, H), lambda e: (0, 0)),
            pl.BlockSpec((1, H), lambda e: (0, 0)),
        ],
        out_specs=pl.BlockSpec((Eb, H), lambda e: (e, 0)),
        out_shape=jax.ShapeDtypeStruct((E, H), jnp.float32),
        scratch_shapes=[
            pltpu.VMEM((2 * H, H), jnp.float32),
            pltpu.VMEM((H, H), jnp.float32),
        ],
    )(xi, xj, rbf, embp, W, bias2, b2)


def _scatter_sc(h, rbf_out, i, zeros_nh):
    E, H = h.shape
    Npad = zeros_nh.shape[0]
    EW = E // NW
    CC = 80
    assert EW % CC == 0 and CC % 8 == 0
    assert Npad % (NS * 8) == 0
    RPT = Npad // NS
    mesh = plsc.VectorSubcoreMesh(core_axis_name="c", subcore_axis_name="s")

    @functools.partial(
        pl.kernel,
        mesh=mesh,
        out_type=(
            jax.ShapeDtypeStruct((Npad, H), jnp.float32),
            jax.ShapeDtypeStruct((Npad, H), jnp.float32),
        ),
        scratch_types=[
            pltpu.VMEM_SHARED((Npad, H), jnp.float32),
            pltpu.VMEM((CC, H), jnp.float32),
            pltpu.VMEM((CC, H), jnp.float32),
            pltpu.VMEM((CC, H), jnp.float32),
            pltpu.VMEM((CC, H), jnp.float32),
            pltpu.VMEM((CC,), jnp.int32),
            pltpu.VMEM((CC,), jnp.int32),
            pltpu.SemaphoreType.DMA,
            pltpu.SemaphoreType.DMA,
            pltpu.SemaphoreType.DMA,
            pltpu.SemaphoreType.DMA,
            pltpu.SemaphoreType.DMA,
            pltpu.SemaphoreType.DMA,
        ],
    )
    def k(h_hbm, ro_hbm, i_hbm, z_hbm, p0_hbm, p1_hbm, acc,
          hb0, hb1, rb0, rb1, ib0, ib1, hs0, hs1, rs0, rs1, is0, is1):
        c = lax.axis_index("c")
        s = lax.axis_index("s")
        w = s * NC + c
        rowsl = pl.ds(s * RPT, RPT)
        pltpu.sync_copy(z_hbm.at[rowsl], acc.at[rowsl])
        plsc.subcore_barrier()

        NCH = EW // CC
        slots = ((hb0, rb0, ib0, hs0, rs0, is0), (hb1, rb1, ib1, hs1, rs1, is1))

        def issue(m, slot):
            hb, rb, ib, hs, rs, isem = slot
            base = w * EW + m * CC
            pltpu.async_copy(h_hbm.at[pl.ds(base, CC)], hb, hs)
            pltpu.async_copy(ro_hbm.at[pl.ds(base, CC)], rb, rs)
            pltpu.async_copy(i_hbm.at[pl.ds(base, CC)], ib, isem)

        def drain(m, slot):
            hb, rb, ib, hs, rs, isem = slot
            base = w * EW + m * CC
            pltpu.make_async_copy(h_hbm.at[pl.ds(base, CC)], hb, hs).wait()
            pltpu.make_async_copy(ro_hbm.at[pl.ds(base, CC)], rb, rs).wait()
            pltpu.make_async_copy(i_hbm.at[pl.ds(base, CC)], ib, isem).wait()

        def process(m, slot, with_issue):
            hb, rb, ib = slot[0], slot[1], slot[2]
            drain(m, slot)

            @plsc.parallel_loop(0, CC, unroll=8)
            def _mul(r):
                for q in range(H // 16):
                    sl = pl.ds(q * 16, 16)
                    hb[r, sl] = hb[r, sl] * rb[r, sl]

            pltpu.sync_copy(hb, acc.at[ib], add=True)
            if with_issue:
                @pl.when(m <= NCH - 3)
                def _():
                    issue(m + 2, slot)

        issue(0, slots[0])
        issue(1, slots[1])

        def pair(g, _):
            process(2 * g, slots[0], True)
            process(2 * g + 1, slots[1], True)
            return 0

        lax.fori_loop(0, NCH // 2, pair, 0)
        if NCH % 2:
            process(NCH - 1, slots[0], False)
        plsc.subcore_barrier()

        @pl.when(c == 0)
        def _w0():
            pltpu.sync_copy(acc.at[rowsl], p0_hbm.at[rowsl])

        @pl.when(c == 1)
        def _w1():
            pltpu.sync_copy(acc.at[rowsl], p1_hbm.at[rowsl])

    return k(h, rbf_out, i, zeros_nh)


def _combine_tc(p0, p1, N):
    H = p0.shape[1]
    Rb = 2000
    assert N % Rb == 0

    def body(a_ref, b_ref, o_ref):
        o_ref[...] = a_ref[...] + b_ref[...]

    return pl.pallas_call(
        body,
        grid=(N // Rb,),
        in_specs=[
            pl.BlockSpec((Rb, H), lambda r: (r, 0)),
            pl.BlockSpec((Rb, H), lambda r: (r, 0)),
        ],
        out_specs=pl.BlockSpec((Rb, H), lambda r: (r, 0)),
        out_shape=jax.ShapeDtypeStruct((N, H), jnp.float32),
    )(p0, p1)


def kernel(x, rbf, i, j, rbf_out, num_nodes, emb, bias, W, b):
    del num_nodes
    N = x.shape[0]
    E = i.shape[0]
    H = rbf.shape[1]
    x = x.astype(jnp.int32)
    i = i.astype(jnp.int32)
    j = j.astype(jnp.int32)
    xi, xj = _type_gather(x, i, j)
    embp = jnp.zeros((H, H), jnp.float32).at[: emb.shape[0]].set(emb)
    h = _edge_tc(xi, xj, rbf, embp, W,
                 bias.reshape(1, H), b.reshape(1, H))
    npad = ((N + NS * 8 - 1) // (NS * 8)) * (NS * 8)
    zeros_nh = jnp.zeros((npad, H), jnp.float32)
    p0, p1 = _scatter_sc(h, rbf_out, i, zeros_nh)
    x_out = _combine_tc(p0, p1, N)
    return (h, x_out)
